# baseline (speedup 1.0000x reference)
import functools

import jax
import jax.numpy as jnp
from jax import lax
from jax.experimental import pallas as pl
from jax.experimental.pallas import tpu as pltpu
from jax.experimental.pallas import tpu_sc as plsc

_NUM_CORES = 2
_NUM_SUBCORES = 16
_NUM_WORKERS = _NUM_CORES * _NUM_SUBCORES

_DIM = 64
_CHUNK = 128


@functools.cache
def _build(B):
  assert B % (_NUM_WORKERS * _CHUNK) == 0
  b_per_w = B // _NUM_WORKERS
  n_chunks = b_per_w // _CHUNK
  mesh = plsc.VectorSubcoreMesh(core_axis_name="c", subcore_axis_name="s")

  @functools.partial(
      pl.kernel,
      out_type=jax.ShapeDtypeStruct((B, _DIM), jnp.float32),
      mesh=mesh,
      scratch_types=[
          pltpu.VMEM((n_chunks, _CHUNK), jnp.int32),
          pltpu.VMEM((_CHUNK, _DIM), jnp.float32),
          pltpu.SemaphoreType.DMA,
      ],
      compiler_params=pltpu.CompilerParams(use_tc_tiling_on_sc=False),
  )
  def k(idx_hbm, table_hbm, out_hbm, idx_v, rows_v, sem):
    wid = lax.axis_index("s") * _NUM_CORES + lax.axis_index("c")
    base = wid * b_per_w
    pltpu.sync_copy(idx_hbm.at[wid], idx_v)

    def body(c, carry):
      pltpu.async_copy(table_hbm.at[idx_v.at[c]], rows_v, sem).wait()
      pltpu.sync_copy(rows_v, out_hbm.at[pl.ds(base + c * _CHUNK, _CHUNK)])
      return carry

    lax.fori_loop(0, n_chunks, body, 0)

  return k


def kernel(token_ids, embeddings):
  rows, cols = token_ids.shape
  B = rows * cols
  idx = token_ids.astype(jnp.int32).reshape(
      _NUM_WORKERS, B // (_NUM_WORKERS * _CHUNK), _CHUNK)
  out = _build(B)(idx, embeddings)
  return out.reshape(rows, cols, _DIM)


# speedup vs baseline: 1.0765x; 1.0765x over previous
import functools

import jax
import jax.numpy as jnp
from jax import lax
from jax.experimental import pallas as pl
from jax.experimental.pallas import tpu as pltpu
from jax.experimental.pallas import tpu_sc as plsc

_NUM_CORES = 2
_NUM_SUBCORES = 16
_NUM_WORKERS = _NUM_CORES * _NUM_SUBCORES

_DIM = 64
_CHUNK = 128


_NBUF = 8
_LAG = 4


@functools.cache
def _build(B):
  assert B % (_NUM_WORKERS * _CHUNK) == 0
  b_per_w = B // _NUM_WORKERS
  n_chunks = b_per_w // _CHUNK
  mesh = plsc.VectorSubcoreMesh(core_axis_name="c", subcore_axis_name="s")

  @functools.partial(
      pl.kernel,
      out_type=jax.ShapeDtypeStruct((B, _DIM), jnp.float32),
      mesh=mesh,
      scratch_types=[
          pltpu.VMEM((n_chunks, _CHUNK), jnp.int32),
          pltpu.VMEM((_NBUF, _CHUNK, _DIM), jnp.float32),
          pltpu.SemaphoreType.DMA((_NBUF,)),
          pltpu.SemaphoreType.DMA((_NBUF,)),
      ],
      compiler_params=pltpu.CompilerParams(use_tc_tiling_on_sc=False),
  )
  def k(idx_hbm, table_hbm, out_hbm, idx_v, rows_v, gsem, wsem):
    wid = lax.axis_index("s") * _NUM_CORES + lax.axis_index("c")
    base = wid * b_per_w
    pltpu.sync_copy(idx_hbm.at[wid], idx_v)

    def step(c, carry):
      b = lax.rem(c, _NBUF)

      @pl.when(c < n_chunks)
      def _issue_gather():
        @pl.when(c >= _NBUF)
        def _():
          pltpu.make_async_copy(
              rows_v.at[b], out_hbm.at[pl.ds(base, _CHUNK)], wsem.at[b]
          ).wait()
        pltpu.async_copy(table_hbm.at[idx_v.at[c]], rows_v.at[b], gsem.at[b])

      @pl.when(c >= _LAG)
      def _retire():
        cc = c - _LAG
        bb = lax.rem(cc, _NBUF)
        pltpu.make_async_copy(
            table_hbm.at[idx_v.at[cc]], rows_v.at[bb], gsem.at[bb]
        ).wait()
        pltpu.async_copy(
            rows_v.at[bb], out_hbm.at[pl.ds(base + cc * _CHUNK, _CHUNK)],
            wsem.at[bb])

      return carry

    lax.fori_loop(0, n_chunks + _LAG, step, 0)

    for b in range(_NBUF):
      pltpu.make_async_copy(
          rows_v.at[b], out_hbm.at[pl.ds(base, _CHUNK)], wsem.at[b]
      ).wait()

  return k


def kernel(token_ids, embeddings):
  rows, cols = token_ids.shape
  B = rows * cols
  idx = token_ids.astype(jnp.int32).reshape(
      _NUM_WORKERS, B // (_NUM_WORKERS * _CHUNK), _CHUNK)
  out = _build(B)(idx, embeddings)
  return out.reshape(rows, cols, _DIM)
